# baseline (speedup 1.0000x reference)
import functools

import jax
import jax.numpy as jnp
from jax import lax
from jax.experimental import pallas as pl
from jax.experimental.pallas import tpu as pltpu
from jax.experimental.pallas import tpu_sc as plsc

IOU_THRESHOLD = 0.5
CONFIDENCE_THRESHOLD = 0.5
MAX_DETECTIONS = 100
NEG_INF = float("-inf")

_SC_WORKERS = 32
_SC_ROWS = 1024
_SC_RPW = _SC_ROWS // _SC_WORKERS
_C = 80


def _nms_body(x1_ref, y1_ref, x2_ref, y2_ref, logits_ref,
              obx_ref, oby_ref, obw_ref, obh_ref, oconf_ref,
              oidx_ref, oval_ref):
    B, N = x1_ref.shape
    x1 = x1_ref[...]
    y1 = y1_ref[...]
    w = x2_ref[...] - x1
    h = y2_ref[...] - y1
    a_min = jnp.minimum(x1, w)
    a_max = jnp.maximum(x1, w)
    b_min = jnp.minimum(y1, h)
    b_max = jnp.maximum(y1, h)
    areas = (a_max - a_min) * (b_max - b_min)

    parts = []
    for b in range(B):
        red = jnp.max(logits_ref[b], axis=-1, keepdims=True)
        parts.append(jnp.transpose(red, (1, 0)))
    conf = jax.nn.sigmoid(jnp.concatenate(parts, axis=0))

    iota = lax.broadcasted_iota(jnp.int32, (B, N), 1)
    boff = lax.broadcasted_iota(jnp.int32, (B, 1), 0) * N
    masked0 = jnp.where(conf > CONFIDENCE_THRESHOLD, conf, NEG_INF)

    def step(t, masked):
        maxval = jnp.max(masked, axis=1, keepdims=True)
        elig = masked == maxval
        idx = jnp.min(jnp.where(elig, iota, N), axis=1, keepdims=True)
        onehot = iota == idx
        valid = maxval > CONFIDENCE_THRESHOLD
        v = valid.astype(jnp.float32)

        def sel(arr):
            return jnp.sum(jnp.where(onehot, arr, 0.0), axis=1, keepdims=True)

        sx1 = sel(x1)
        sy1 = sel(y1)
        sw = sel(w)
        sh = sel(h)
        samin = jnp.minimum(sx1, sw)
        samax = jnp.maximum(sx1, sw)
        sbmin = jnp.minimum(sy1, sh)
        sbmax = jnp.maximum(sy1, sh)
        sarea = (samax - samin) * (sbmax - sbmin)

        inter_a = jnp.maximum(0.0, jnp.minimum(samax, a_max) - jnp.maximum(samin, a_min))
        inter_b = jnp.maximum(0.0, jnp.minimum(sbmax, b_max) - jnp.maximum(sbmin, b_min))
        inter = inter_a * inter_b
        union = sarea + areas - inter
        denom = jnp.where(union > 0.0, union, 1.0)
        iou = jnp.where(union > 0.0, inter / denom, 0.0)
        suppress = iou > IOU_THRESHOLD

        new_masked = jnp.where(suppress | onehot | (~valid), NEG_INF, masked)

        obx_ref[pl.ds(t, 1), :] = (sx1 * v).reshape(1, B)
        oby_ref[pl.ds(t, 1), :] = (sy1 * v).reshape(1, B)
        obw_ref[pl.ds(t, 1), :] = (sw * v).reshape(1, B)
        obh_ref[pl.ds(t, 1), :] = (sh * v).reshape(1, B)
        oconf_ref[pl.ds(t, 1), :] = jnp.where(valid, maxval, 0.0).reshape(1, B)
        oidx_ref[pl.ds(t, 1), :] = (idx + boff).reshape(1, B)
        oval_ref[pl.ds(t, 1), :] = v.reshape(1, B)
        return new_masked

    lax.fori_loop(0, MAX_DETECTIONS, step, masked0)


@functools.partial(
    pl.kernel,
    out_type=jax.ShapeDtypeStruct((_SC_ROWS, _C), jnp.float32),
    mesh=plsc.VectorSubcoreMesh(core_axis_name="c", subcore_axis_name="s"),
    scratch_types=[
        pltpu.VMEM((_SC_RPW,), jnp.int32),
        pltpu.VMEM((_SC_RPW, _C), jnp.float32),
        pltpu.VMEM((_SC_RPW, _C), jnp.float32),
        pltpu.SemaphoreType.DMA,
    ],
    compiler_params=pltpu.CompilerParams(use_tc_tiling_on_sc=False),
)
def _sc_gather_softmax(table_hbm, idx_hbm, out_hbm, idx_v, rows_v, prob_v, sem):
    wid = lax.axis_index("s") * 2 + lax.axis_index("c")
    base = wid * _SC_RPW
    pltpu.sync_copy(idx_hbm.at[pl.ds(base, _SC_RPW)], idx_v)
    pltpu.async_copy(table_hbm.at[idx_v], rows_v, sem).wait()

    iota16 = lax.iota(jnp.int32, 16)
    gdn = lax.GatherDimensionNumbers(
        offset_dims=(), collapsed_slice_dims=(0,), start_index_map=(0,))

    def _shuf(x, idx):
        return lax.gather(x, idx[:, None], gdn, (1,),
                          mode=lax.GatherScatterMode.PROMISE_IN_BOUNDS)

    def _allreduce(x, op):
        for stp in (8, 4, 2, 1):
            x = op(x, _shuf(x, jnp.bitwise_xor(iota16, stp)))
        return x

    for r in range(_SC_RPW):
        vs = [rows_v[r, pl.ds(16 * j, 16)] for j in range(5)]
        m01 = jnp.maximum(vs[0], vs[1])
        m23 = jnp.maximum(vs[2], vs[3])
        m = jnp.maximum(jnp.maximum(m01, m23), vs[4])
        ms = _allreduce(m, jnp.maximum)
        es = [jnp.exp(vj - ms) for vj in vs]
        s = (es[0] + es[1]) + (es[2] + es[3]) + es[4]
        ss = _allreduce(s, jnp.add)
        for j in range(5):
            prob_v[r, pl.ds(16 * j, 16)] = es[j] / ss
    pltpu.sync_copy(prob_v, out_hbm.at[pl.ds(base, _SC_RPW)])


def kernel(box_prediction, class_prediction):
    B, N, C = class_prediction.shape
    x1 = box_prediction[..., 0]
    y1 = box_prediction[..., 1]
    x2 = box_prediction[..., 2]
    y2 = box_prediction[..., 3]

    out_shapes = (
        jax.ShapeDtypeStruct((MAX_DETECTIONS, B), jnp.float32),
        jax.ShapeDtypeStruct((MAX_DETECTIONS, B), jnp.float32),
        jax.ShapeDtypeStruct((MAX_DETECTIONS, B), jnp.float32),
        jax.ShapeDtypeStruct((MAX_DETECTIONS, B), jnp.float32),
        jax.ShapeDtypeStruct((MAX_DETECTIONS, B), jnp.float32),
        jax.ShapeDtypeStruct((MAX_DETECTIONS, B), jnp.int32),
        jax.ShapeDtypeStruct((MAX_DETECTIONS, B), jnp.float32),
    )
    bx, by, bw, bh, cf, fidx, val = pl.pallas_call(
        _nms_body,
        out_shape=out_shapes,
    )(x1, y1, x2, y2, class_prediction)

    table = class_prediction.reshape(B * N, C)
    idx_flat = fidx.T.reshape(B * MAX_DETECTIONS)
    idx_pad = jnp.concatenate(
        [idx_flat, jnp.zeros((_SC_ROWS - B * MAX_DETECTIONS,), jnp.int32)])
    probs = _sc_gather_softmax(table, idx_pad)
    cls = probs[:B * MAX_DETECTIONS].reshape(B, MAX_DETECTIONS, C)
    cls = cls * val.T.reshape(B, MAX_DETECTIONS, 1)

    nms_box = jnp.stack([bx.T, by.T, bw.T, bh.T], axis=-1)
    nms_conf = cf.T
    return nms_box, cls, nms_conf


# speedup vs baseline: 1.3208x; 1.3208x over previous
import functools

import jax
import jax.numpy as jnp
from jax import lax
from jax.experimental import pallas as pl
from jax.experimental.pallas import tpu as pltpu

IOU_THRESHOLD = 0.5
CONFIDENCE_THRESHOLD = 0.5
MAX_DETECTIONS = 100
NEG_INF = float("-inf")


def _nms_body(x1_ref, y1_ref, x2_ref, y2_ref, logits_ref,
              obx_ref, oby_ref, obw_ref, obh_ref, oconf_ref, ocls_ref,
              idx_scr, val_scr):
    B, N = x1_ref.shape
    x1 = x1_ref[...]
    y1 = y1_ref[...]
    w = x2_ref[...] - x1
    h = y2_ref[...] - y1
    a_min = jnp.minimum(x1, w)
    a_max = jnp.maximum(x1, w)
    b_min = jnp.minimum(y1, h)
    b_max = jnp.maximum(y1, h)
    areas = (a_max - a_min) * (b_max - b_min)

    parts = []
    for b in range(B):
        red = jnp.max(logits_ref[b], axis=-1, keepdims=True)
        parts.append(jnp.transpose(red, (1, 0)))
    conf = jax.nn.sigmoid(jnp.concatenate(parts, axis=0))

    iota = lax.broadcasted_iota(jnp.int32, (B, N), 1)
    masked0 = jnp.where(conf > CONFIDENCE_THRESHOLD, conf, NEG_INF)

    def step(t, masked):
        maxval = jnp.max(masked, axis=1, keepdims=True)
        elig = masked == maxval
        idx = jnp.min(jnp.where(elig, iota, N), axis=1, keepdims=True)
        onehot = iota == idx
        valid = maxval > CONFIDENCE_THRESHOLD
        v = valid.astype(jnp.float32)

        def sel(arr):
            return jnp.sum(jnp.where(onehot, arr, 0.0), axis=1, keepdims=True)

        sx1 = sel(x1)
        sy1 = sel(y1)
        sw = sel(w)
        sh = sel(h)
        samin = jnp.minimum(sx1, sw)
        samax = jnp.maximum(sx1, sw)
        sbmin = jnp.minimum(sy1, sh)
        sbmax = jnp.maximum(sy1, sh)
        sarea = (samax - samin) * (sbmax - sbmin)

        inter_a = jnp.maximum(0.0, jnp.minimum(samax, a_max) - jnp.maximum(samin, a_min))
        inter_b = jnp.maximum(0.0, jnp.minimum(sbmax, b_max) - jnp.maximum(sbmin, b_min))
        inter = inter_a * inter_b
        union = sarea + areas - inter
        denom = jnp.where(union > 0.0, union, 1.0)
        iou = jnp.where(union > 0.0, inter / denom, 0.0)
        suppress = iou > IOU_THRESHOLD

        new_masked = jnp.where(suppress | onehot | (~valid), NEG_INF, masked)

        obx_ref[pl.ds(t, 1), :] = (sx1 * v).reshape(1, B)
        oby_ref[pl.ds(t, 1), :] = (sy1 * v).reshape(1, B)
        obw_ref[pl.ds(t, 1), :] = (sw * v).reshape(1, B)
        obh_ref[pl.ds(t, 1), :] = (sh * v).reshape(1, B)
        oconf_ref[pl.ds(t, 1), :] = jnp.where(valid, maxval, 0.0).reshape(1, B)
        idx_scr[pl.ds(t, 1), :] = idx.astype(jnp.int32).reshape(1, B)
        val_scr[pl.ds(t, 1), :] = v.reshape(1, B)
        return new_masked

    lax.fori_loop(0, MAX_DETECTIONS, step, masked0, unroll=4)

    iota_n = lax.broadcasted_iota(jnp.int32, (MAX_DETECTIONS, N), 1)
    for b in range(B):
        idx_b = idx_scr[:, b].reshape(MAX_DETECTIONS, 1)
        onehot_b = (iota_n == idx_b).astype(jnp.bfloat16)
        lg = logits_ref[b]
        hi = lg.astype(jnp.bfloat16)
        r1 = lg - hi.astype(jnp.float32)
        mid = r1.astype(jnp.bfloat16)
        lo = (r1 - mid.astype(jnp.float32)).astype(jnp.bfloat16)
        rows = (jnp.dot(onehot_b, hi, preferred_element_type=jnp.float32)
                + jnp.dot(onehot_b, mid, preferred_element_type=jnp.float32)
                + jnp.dot(onehot_b, lo, preferred_element_type=jnp.float32))
        m = jnp.max(rows, axis=1, keepdims=True)
        e = jnp.exp(rows - m)
        p = e / jnp.sum(e, axis=1, keepdims=True)
        ocls_ref[b] = p * val_scr[:, b].reshape(MAX_DETECTIONS, 1)


@functools.partial(jax.jit, static_argnames=("interpret",))
def kernel(box_prediction, class_prediction, interpret=False):
    B, N, C = class_prediction.shape
    x1 = box_prediction[..., 0]
    y1 = box_prediction[..., 1]
    x2 = box_prediction[..., 2]
    y2 = box_prediction[..., 3]

    out_shapes = (
        jax.ShapeDtypeStruct((MAX_DETECTIONS, B), jnp.float32),
        jax.ShapeDtypeStruct((MAX_DETECTIONS, B), jnp.float32),
        jax.ShapeDtypeStruct((MAX_DETECTIONS, B), jnp.float32),
        jax.ShapeDtypeStruct((MAX_DETECTIONS, B), jnp.float32),
        jax.ShapeDtypeStruct((MAX_DETECTIONS, B), jnp.float32),
        jax.ShapeDtypeStruct((B, MAX_DETECTIONS, C), jnp.float32),
    )
    bx, by, bw, bh, cf, cls = pl.pallas_call(
        _nms_body,
        out_shape=out_shapes,
        scratch_shapes=[
            pltpu.VMEM((MAX_DETECTIONS, B), jnp.int32),
            pltpu.VMEM((MAX_DETECTIONS, B), jnp.float32),
        ],
        interpret=interpret,
    )(x1, y1, x2, y2, class_prediction)

    nms_box = jnp.stack([bx.T, by.T, bw.T, bh.T], axis=-1)
    nms_conf = cf.T
    return nms_box, cls, nms_conf
